# Initial kernel scaffold; baseline (speedup 1.0000x reference)
import jax
import jax.numpy as jnp
from jax.experimental import pallas as pl


def kernel(state, edge_index, edge_attributes, batch, W, b):
    raise NotImplementedError("write your pallas kernel here")


# speedup vs baseline: 2.3942x; 2.3942x over previous
import functools

import jax
import jax.numpy as jnp
from jax import lax
from jax.experimental import pallas as pl
from jax.experimental.pallas import tpu as pltpu
from jax.experimental.pallas import tpu_sc as plsc

N, E, D = 10000, 160000, 256
N2 = 10240
NRD = N2 + 8
NSC = 2
NTILE = 16
CHUNK = 2048
EPT = 10240
E_PAD = EPT * NTILE
CHUNK_A = 1024
EPT_A = E_PAD // (NSC * NTILE)
SH_DEG = 10496
DEG_SLICE = SH_DEG // NTILE
DPP = 32
NPASS = 4
DPT = DPP // NTILE


def _deg_body(dst_hbm, degp_hbm, dst_v, ones_v, stage_v, sem, deg_s):
    c = lax.axis_index("c")
    s = lax.axis_index("s")

    for i in range(CHUNK_A // 16):
        ones_v[pl.ds(16 * i, 16)] = jnp.full((16,), 1.0, jnp.float32)
    for i in range(DEG_SLICE // 16):
        stage_v[pl.ds(16 * i, 16)] = jnp.zeros((16,), jnp.float32)
    pltpu.sync_copy(stage_v, deg_s.at[pl.ds(s * DEG_SLICE, DEG_SLICE)])
    plsc.subcore_barrier()

    base = (s * NSC + c) * EPT_A

    def body(ch, carry):
        e0 = pl.multiple_of(base + ch * CHUNK_A, CHUNK_A)
        pltpu.sync_copy(dst_hbm.at[pl.ds(e0, CHUNK_A)], dst_v)
        pltpu.async_copy(ones_v, deg_s.at[dst_v], sem, add=True).wait()
        return carry

    lax.fori_loop(0, EPT_A // CHUNK_A, body, 0)
    plsc.subcore_barrier()
    pltpu.sync_copy(deg_s.at[pl.ds(s * DEG_SLICE, DEG_SLICE)], stage_v)
    pltpu.sync_copy(
        stage_v, degp_hbm.at[pl.ds(c * SH_DEG + s * DEG_SLICE, DEG_SLICE)]
    )


@functools.lru_cache(maxsize=None)
def _deg_kernel_fn():
    mesh = plsc.VectorSubcoreMesh(core_axis_name="c", subcore_axis_name="s")
    return pl.kernel(
        _deg_body,
        mesh=mesh,
        out_type=jax.ShapeDtypeStruct((NSC * SH_DEG,), jnp.float32),
        scratch_types=[
            pltpu.VMEM((CHUNK_A,), jnp.int32),
            pltpu.VMEM((CHUNK_A,), jnp.float32),
            pltpu.VMEM((DEG_SLICE,), jnp.float32),
            pltpu.SemaphoreType.DMA,
            pltpu.MemorySpace.VMEM_SHARED((SH_DEG,), jnp.float32),
        ],
    )


def _mm_body(state_ref, w_ref, degp_ref, yt_ref, dinv_ref):
    x_t = lax.dot_general(
        w_ref[...], state_ref[...],
        dimension_numbers=(((0,), (1,)), ((), ())),
        preferred_element_type=jnp.float32,
    )
    deg = 1.0 + degp_ref[0:1, :] + degp_ref[1:2, :]
    dinv = lax.rsqrt(deg)
    yt_ref[...] = x_t * dinv
    dinv_ref[...] = dinv


def _matmul_scale(state_pad, W, degp):
    BR = 1024
    return pl.pallas_call(
        _mm_body,
        grid=(N2 // BR,),
        in_specs=[
            pl.BlockSpec((BR, D), lambda i: (i, 0)),
            pl.BlockSpec((D, D), lambda i: (0, 0)),
            pl.BlockSpec((2, BR), lambda i: (0, i)),
        ],
        out_specs=[
            pl.BlockSpec((D, BR), lambda i: (0, i)),
            pl.BlockSpec((1, BR), lambda i: (0, i)),
        ],
        out_shape=[
            jax.ShapeDtypeStruct((D, N2), jnp.float32),
            jax.ShapeDtypeStruct((1, N2), jnp.float32),
        ],
    )(state_pad, W, degp)


def _edge_body(y_hbm, src_hbm, dst_hbm, acc_hbm,
               src_v, dst_v, eb0, eb1, eb2, eb3, stage_v, gsem, ssem,
               y_q, acc_q):
    c = lax.axis_index("c")
    s = lax.axis_index("s")
    ebufs = (eb0, eb1, eb2, eb3)

    for p in range(NPASS):
        gd0 = c * (DPP * NPASS) + p * DPP + s * DPT
        hb = pl.multiple_of(gd0 * N2, 8)
        pltpu.sync_copy(y_hbm.at[pl.ds(hb, DPT * N2)], stage_v)
        pltpu.sync_copy(stage_v, y_q.at[pl.ds(s * DPT * N2, DPT * N2)])
        for jj in range(DPT):
            pltpu.sync_copy(
                stage_v.at[pl.ds(jj * N2, N2)],
                acc_q.at[pl.ds((s * DPT + jj) * NRD, N2)],
            )
        plsc.subcore_barrier()

        def chunk_body(ch, carry):
            e0 = pl.multiple_of(s * EPT + ch * CHUNK, CHUNK)
            pltpu.sync_copy(src_hbm.at[pl.ds(e0, CHUNK)], src_v)
            pltpu.sync_copy(dst_hbm.at[pl.ds(e0, CHUNK)], dst_v)

            def dim_group(g, carry2):
                descs = []
                for j2 in range(4):
                    j = g * 4 + j2
                    yb = pl.multiple_of(j * N2, 8)
                    descs.append(
                        pltpu.async_copy(
                            y_q.at[pl.ds(yb, N2)].at[src_v], ebufs[j2], gsem
                        )
                    )
                for d_ in descs:
                    d_.wait()
                descs = []
                for j2 in range(4):
                    j = g * 4 + j2
                    ab = pl.multiple_of(j * NRD, 8)
                    descs.append(
                        pltpu.async_copy(
                            ebufs[j2], acc_q.at[pl.ds(ab, NRD)].at[dst_v],
                            ssem, add=True,
                        )
                    )
                for d_ in descs:
                    d_.wait()
                return carry2

            lax.fori_loop(0, DPP // 4, dim_group, 0)
            return carry

        lax.fori_loop(0, EPT // CHUNK, chunk_body, 0)
        plsc.subcore_barrier()

        for jj in range(DPT):
            pltpu.sync_copy(
                acc_q.at[pl.ds((s * DPT + jj) * NRD, N2)],
                stage_v.at[pl.ds(jj * N2, N2)],
            )
        pltpu.sync_copy(stage_v, acc_hbm.at[pl.ds(hb, DPT * N2)])
        plsc.subcore_barrier()


@functools.lru_cache(maxsize=None)
def _edge_kernel_fn():
    mesh = plsc.VectorSubcoreMesh(core_axis_name="c", subcore_axis_name="s")
    return pl.kernel(
        _edge_body,
        mesh=mesh,
        out_type=jax.ShapeDtypeStruct((D * N2,), jnp.float32),
        scratch_types=[
            pltpu.VMEM((CHUNK,), jnp.int32),
            pltpu.VMEM((CHUNK,), jnp.int32),
            pltpu.VMEM((CHUNK,), jnp.float32),
            pltpu.VMEM((CHUNK,), jnp.float32),
            pltpu.VMEM((CHUNK,), jnp.float32),
            pltpu.VMEM((CHUNK,), jnp.float32),
            pltpu.VMEM((DPT * N2,), jnp.float32),
            pltpu.SemaphoreType.DMA,
            pltpu.SemaphoreType.DMA,
            pltpu.MemorySpace.VMEM_SHARED((DPP * N2,), jnp.float32),
            pltpu.MemorySpace.VMEM_SHARED((DPP * NRD,), jnp.float32),
        ],
    )


def _fin_body(acc_ref, dinv_ref, b_ref, out_ref):
    out_ref[...] = jnp.maximum(
        acc_ref[...] * dinv_ref[...] + b_ref[...], 0.0
    ).T


def _finish(acc_t, dinv_row, b_col):
    BR = 1024
    return pl.pallas_call(
        _fin_body,
        grid=(N2 // BR,),
        in_specs=[
            pl.BlockSpec((D, BR), lambda i: (0, i)),
            pl.BlockSpec((1, BR), lambda i: (0, i)),
            pl.BlockSpec((D, 1), lambda i: (0, 0)),
        ],
        out_specs=pl.BlockSpec((BR, D), lambda i: (i, 0)),
        out_shape=jax.ShapeDtypeStruct((N2, D), jnp.float32),
    )(acc_t, dinv_row, b_col)


def kernel(state, edge_index, edge_attributes, batch, W, b):
    src = edge_index[0]
    dst = edge_index[1]
    pad_e = E_PAD - E
    src_pad = jnp.concatenate([src, jnp.zeros((pad_e,), jnp.int32)])
    dst_pad = jnp.concatenate([dst, jnp.full((pad_e,), N2, jnp.int32)])
    state_pad = jnp.concatenate(
        [state, jnp.zeros((N2 - N, D), jnp.float32)], axis=0
    )

    degp = _deg_kernel_fn()(dst_pad).reshape(NSC, SH_DEG)[:, :N2]
    y_t, dinv_row = _matmul_scale(state_pad, W, degp)
    acc_flat = _edge_kernel_fn()(y_t.reshape(D * N2), src_pad, dst_pad)
    out = _finish(acc_flat.reshape(D, N2), dinv_row, b.reshape(D, 1))
    return out[:N]


# speedup vs baseline: 3.0197x; 1.2613x over previous
import functools

import jax
import jax.numpy as jnp
from jax import lax
from jax.experimental import pallas as pl
from jax.experimental.pallas import tpu as pltpu
from jax.experimental.pallas import tpu_sc as plsc

N, E, D = 10000, 160000, 256
N2 = 10240
NRD = N2 + 8
NSC = 2
NTILE = 16
CHUNK = 2048
EPT = 10240
E_PAD = EPT * NTILE
CHUNK_A = 1024
EPT_A = E_PAD // (NSC * NTILE)
SH_DEG = 10496
DEG_SLICE = SH_DEG // NTILE
DPP = 32
NPASS = 4
DPT = DPP // NTILE


def _deg_body(dst_hbm, degp_hbm, dst_v, ones_v, stage_v, sem, deg_s):
    c = lax.axis_index("c")
    s = lax.axis_index("s")

    for i in range(CHUNK_A // 16):
        ones_v[pl.ds(16 * i, 16)] = jnp.full((16,), 1.0, jnp.float32)
    for i in range(DEG_SLICE // 16):
        stage_v[pl.ds(16 * i, 16)] = jnp.zeros((16,), jnp.float32)
    pltpu.sync_copy(stage_v, deg_s.at[pl.ds(s * DEG_SLICE, DEG_SLICE)])
    plsc.subcore_barrier()

    base = (s * NSC + c) * EPT_A

    def body(ch, carry):
        e0 = pl.multiple_of(base + ch * CHUNK_A, CHUNK_A)
        pltpu.sync_copy(dst_hbm.at[pl.ds(e0, CHUNK_A)], dst_v)
        pltpu.async_copy(ones_v, deg_s.at[dst_v], sem, add=True).wait()
        return carry

    lax.fori_loop(0, EPT_A // CHUNK_A, body, 0)
    plsc.subcore_barrier()
    pltpu.sync_copy(deg_s.at[pl.ds(s * DEG_SLICE, DEG_SLICE)], stage_v)
    pltpu.sync_copy(
        stage_v, degp_hbm.at[pl.ds(c * SH_DEG + s * DEG_SLICE, DEG_SLICE)]
    )


@functools.lru_cache(maxsize=None)
def _deg_kernel_fn():
    mesh = plsc.VectorSubcoreMesh(core_axis_name="c", subcore_axis_name="s")
    return pl.kernel(
        _deg_body,
        mesh=mesh,
        out_type=jax.ShapeDtypeStruct((NSC * SH_DEG,), jnp.float32),
        scratch_types=[
            pltpu.VMEM((CHUNK_A,), jnp.int32),
            pltpu.VMEM((CHUNK_A,), jnp.float32),
            pltpu.VMEM((DEG_SLICE,), jnp.float32),
            pltpu.SemaphoreType.DMA,
            pltpu.MemorySpace.VMEM_SHARED((SH_DEG,), jnp.float32),
        ],
    )


def _mm_body(state_ref, w_ref, degp_ref, yt_ref, dinv_ref):
    x_t = lax.dot_general(
        w_ref[...], state_ref[...],
        dimension_numbers=(((0,), (1,)), ((), ())),
        preferred_element_type=jnp.float32,
    )
    deg = 1.0 + degp_ref[0:1, :] + degp_ref[1:2, :]
    dinv = lax.rsqrt(deg)
    yt_ref[...] = x_t * dinv
    dinv_ref[...] = dinv


def _matmul_scale(state_pad, W, degp):
    BR = 1024
    return pl.pallas_call(
        _mm_body,
        grid=(N2 // BR,),
        in_specs=[
            pl.BlockSpec((BR, D), lambda i: (i, 0)),
            pl.BlockSpec((D, D), lambda i: (0, 0)),
            pl.BlockSpec((2, BR), lambda i: (0, i)),
        ],
        out_specs=[
            pl.BlockSpec((D, BR), lambda i: (0, i)),
            pl.BlockSpec((1, BR), lambda i: (0, i)),
        ],
        out_shape=[
            jax.ShapeDtypeStruct((D, N2), jnp.float32),
            jax.ShapeDtypeStruct((1, N2), jnp.float32),
        ],
    )(state_pad, W, degp)


def _edge_body(y_hbm, src_hbm, dst_hbm, acc_hbm,
               src_v, dst_v, eb0, eb1, eb2, eb3, eb4, eb5, eb6, eb7,
               stage_v, gsem, ssem, y_q, acc_q):
    c = lax.axis_index("c")
    s = lax.axis_index("s")
    bufsets = ((eb0, eb1, eb2, eb3), (eb4, eb5, eb6, eb7))

    for p in range(NPASS):
        gd0 = c * (DPP * NPASS) + p * DPP + s * DPT
        hb = pl.multiple_of(gd0 * N2, 8)
        pltpu.sync_copy(y_hbm.at[pl.ds(hb, DPT * N2)], stage_v)
        pltpu.sync_copy(stage_v, y_q.at[pl.ds(s * DPT * N2, DPT * N2)])
        for jj in range(DPT):
            pltpu.sync_copy(
                stage_v.at[pl.ds(jj * N2, N2)],
                acc_q.at[pl.ds((s * DPT + jj) * NRD, N2)],
            )
        plsc.subcore_barrier()

        NG = DPP // 4

        def fire_gathers(g, bufs):
            descs = []
            for j2 in range(4):
                yb = pl.multiple_of((g * 4 + j2) * N2, 8)
                descs.append(
                    pltpu.async_copy(
                        y_q.at[pl.ds(yb, N2)].at[src_v], bufs[j2], gsem
                    )
                )
            return descs

        def fire_scatters(g, bufs):
            descs = []
            for j2 in range(4):
                ab = pl.multiple_of((g * 4 + j2) * NRD, 8)
                descs.append(
                    pltpu.async_copy(
                        bufs[j2], acc_q.at[pl.ds(ab, NRD)].at[dst_v],
                        ssem, add=True,
                    )
                )
            return descs

        def chunk_body(ch, carry):
            e0 = pl.multiple_of(s * EPT + ch * CHUNK, CHUNK)
            pltpu.sync_copy(src_hbm.at[pl.ds(e0, CHUNK)], src_v)
            pltpu.sync_copy(dst_hbm.at[pl.ds(e0, CHUNK)], dst_v)

            gd = fire_gathers(0, bufsets[0])
            sd_prev = None
            for g in range(NG):
                for d_ in gd:
                    d_.wait()
                sd = fire_scatters(g, bufsets[g % 2])
                if sd_prev is not None:
                    for d_ in sd_prev:
                        d_.wait()
                if g + 1 < NG:
                    gd = fire_gathers(g + 1, bufsets[(g + 1) % 2])
                sd_prev = sd
            for d_ in sd_prev:
                d_.wait()
            return carry

        lax.fori_loop(0, EPT // CHUNK, chunk_body, 0)
        plsc.subcore_barrier()

        for jj in range(DPT):
            pltpu.sync_copy(
                acc_q.at[pl.ds((s * DPT + jj) * NRD, N2)],
                stage_v.at[pl.ds(jj * N2, N2)],
            )
        pltpu.sync_copy(stage_v, acc_hbm.at[pl.ds(hb, DPT * N2)])
        plsc.subcore_barrier()


@functools.lru_cache(maxsize=None)
def _edge_kernel_fn():
    mesh = plsc.VectorSubcoreMesh(core_axis_name="c", subcore_axis_name="s")
    return pl.kernel(
        _edge_body,
        mesh=mesh,
        out_type=jax.ShapeDtypeStruct((D * N2,), jnp.float32),
        scratch_types=[
            pltpu.VMEM((CHUNK,), jnp.int32),
            pltpu.VMEM((CHUNK,), jnp.int32),
            pltpu.VMEM((CHUNK,), jnp.float32),
            pltpu.VMEM((CHUNK,), jnp.float32),
            pltpu.VMEM((CHUNK,), jnp.float32),
            pltpu.VMEM((CHUNK,), jnp.float32),
            pltpu.VMEM((CHUNK,), jnp.float32),
            pltpu.VMEM((CHUNK,), jnp.float32),
            pltpu.VMEM((CHUNK,), jnp.float32),
            pltpu.VMEM((CHUNK,), jnp.float32),
            pltpu.VMEM((DPT * N2,), jnp.float32),
            pltpu.SemaphoreType.DMA,
            pltpu.SemaphoreType.DMA,
            pltpu.MemorySpace.VMEM_SHARED((DPP * N2,), jnp.float32),
            pltpu.MemorySpace.VMEM_SHARED((DPP * NRD,), jnp.float32),
        ],
    )


def _fin_body(acc_ref, dinv_ref, b_ref, out_ref):
    out_ref[...] = jnp.maximum(
        acc_ref[...] * dinv_ref[...] + b_ref[...], 0.0
    ).T


def _finish(acc_t, dinv_row, b_col):
    BR = 1024
    return pl.pallas_call(
        _fin_body,
        grid=(N2 // BR,),
        in_specs=[
            pl.BlockSpec((D, BR), lambda i: (0, i)),
            pl.BlockSpec((1, BR), lambda i: (0, i)),
            pl.BlockSpec((D, 1), lambda i: (0, 0)),
        ],
        out_specs=pl.BlockSpec((BR, D), lambda i: (i, 0)),
        out_shape=jax.ShapeDtypeStruct((N2, D), jnp.float32),
    )(acc_t, dinv_row, b_col)


def kernel(state, edge_index, edge_attributes, batch, W, b):
    src = edge_index[0]
    dst = edge_index[1]
    pad_e = E_PAD - E
    src_pad = jnp.concatenate([src, jnp.zeros((pad_e,), jnp.int32)])
    dst_pad = jnp.concatenate([dst, jnp.full((pad_e,), N2, jnp.int32)])
    state_pad = jnp.concatenate(
        [state, jnp.zeros((N2 - N, D), jnp.float32)], axis=0
    )

    degp = _deg_kernel_fn()(dst_pad).reshape(NSC, SH_DEG)[:, :N2]
    y_t, dinv_row = _matmul_scale(state_pad, W, degp)
    acc_flat = _edge_kernel_fn()(y_t.reshape(D * N2), src_pad, dst_pad)
    out = _finish(acc_flat.reshape(D, N2), dinv_row, b.reshape(D, 1))
    return out[:N]


# speedup vs baseline: 4.8077x; 1.5921x over previous
import functools

import jax
import jax.numpy as jnp
from jax import lax
from jax.experimental import pallas as pl
from jax.experimental.pallas import tpu as pltpu
from jax.experimental.pallas import tpu_sc as plsc

N, E, D = 10000, 160000, 256
DH = D // 2
N2 = 10240
NSC = 2
NTILE = 16
CH = 320
EPT = 10240
E_PAD = EPT * NTILE
CHUNK_A = 1024
EPT_A = E_PAD // (NSC * NTILE)
SH_DEG = 10496
DEG_SLICE = SH_DEG // NTILE
RSC = N2 // NSC
RPT = RSC // NTILE
SH_ACC = RSC + 8 * NTILE


def _deg_body(dst_hbm, degp_hbm, dst_v, ones_v, stage_v, sem, deg_s):
    c = lax.axis_index("c")
    s = lax.axis_index("s")

    for i in range(CHUNK_A // 16):
        ones_v[pl.ds(16 * i, 16)] = jnp.full((16,), 1.0, jnp.float32)
    for i in range(DEG_SLICE // 16):
        stage_v[pl.ds(16 * i, 16)] = jnp.zeros((16,), jnp.float32)
    pltpu.sync_copy(stage_v, deg_s.at[pl.ds(s * DEG_SLICE, DEG_SLICE)])
    plsc.subcore_barrier()

    base = (s * NSC + c) * EPT_A

    def body(ch, carry):
        e0 = pl.multiple_of(base + ch * CHUNK_A, CHUNK_A)
        pltpu.sync_copy(dst_hbm.at[pl.ds(e0, CHUNK_A)], dst_v)
        pltpu.async_copy(ones_v, deg_s.at[dst_v], sem, add=True).wait()
        return carry

    lax.fori_loop(0, EPT_A // CHUNK_A, body, 0)
    plsc.subcore_barrier()
    pltpu.sync_copy(deg_s.at[pl.ds(s * DEG_SLICE, DEG_SLICE)], stage_v)
    pltpu.sync_copy(
        stage_v, degp_hbm.at[pl.ds(c * SH_DEG + s * DEG_SLICE, DEG_SLICE)]
    )


@functools.lru_cache(maxsize=None)
def _deg_kernel_fn():
    mesh = plsc.VectorSubcoreMesh(core_axis_name="c", subcore_axis_name="s")
    return pl.kernel(
        _deg_body,
        mesh=mesh,
        out_type=jax.ShapeDtypeStruct((NSC * SH_DEG,), jnp.float32),
        scratch_types=[
            pltpu.VMEM((CHUNK_A,), jnp.int32),
            pltpu.VMEM((CHUNK_A,), jnp.float32),
            pltpu.VMEM((DEG_SLICE,), jnp.float32),
            pltpu.SemaphoreType.DMA,
            pltpu.MemorySpace.VMEM_SHARED((SH_DEG,), jnp.float32),
        ],
    )


def _mm_body(state_ref, w_ref, degp_ref, ylo_ref, yhi_ref, dinv_ref):
    x = jnp.dot(state_ref[...], w_ref[...], preferred_element_type=jnp.float32)
    deg = 1.0 + degp_ref[:, 0:1] + degp_ref[:, 1:2]
    dinv = lax.rsqrt(deg)
    y = x * dinv
    ylo_ref[...] = y[:, :DH]
    yhi_ref[...] = y[:, DH:]
    dinv_ref[...] = dinv


def _matmul_scale(state_pad, W, degp_t):
    BR = 1024
    return pl.pallas_call(
        _mm_body,
        grid=(N2 // BR,),
        in_specs=[
            pl.BlockSpec((BR, D), lambda i: (i, 0)),
            pl.BlockSpec((D, D), lambda i: (0, 0)),
            pl.BlockSpec((BR, 2), lambda i: (i, 0)),
        ],
        out_specs=[
            pl.BlockSpec((BR, DH), lambda i: (i, 0)),
            pl.BlockSpec((BR, DH), lambda i: (i, 0)),
            pl.BlockSpec((BR, 1), lambda i: (i, 0)),
        ],
        out_shape=[
            jax.ShapeDtypeStruct((N2, DH), jnp.float32),
            jax.ShapeDtypeStruct((N2, DH), jnp.float32),
            jax.ShapeDtypeStruct((N2, 1), jnp.float32),
        ],
    )(state_pad, W, degp_t)


def _edge_body(ylo_hbm, yhi_hbm, src_hbm, dst_hbm, alo_hbm, ahi_hbm,
               sv0, dv0, av0, sv1, dv1, av1, rows0, rows1,
               gsem, ssem, acc_s):
    c = lax.axis_index("c")
    s = lax.axis_index("s")
    row_base = c * RSC
    iota = lax.iota(jnp.int32, 16)
    dum = RSC + s * 8 + (iota & 7)

    r0 = row_base + s * RPT
    a0 = s * RPT

    def load_idx(e0, sv, dv, av):
        pltpu.sync_copy(src_hbm.at[pl.ds(e0, CH)], sv)
        pltpu.sync_copy(dst_hbm.at[pl.ds(e0, CH)], dv)
        for i in range(CH // 16):
            dd = dv[pl.ds(16 * i, 16)]
            local = dd - row_base
            inr = (local >= 0) & (local < RSC)
            av[pl.ds(16 * i, 16)] = jnp.where(inr, local, dum)

    for y_hbm, acc_hbm in ((ylo_hbm, alo_hbm), (yhi_hbm, ahi_hbm)):
        pltpu.sync_copy(y_hbm.at[pl.ds(r0, RPT)], rows0)
        pltpu.sync_copy(rows0, acc_s.at[pl.ds(a0, RPT)])
        plsc.subcore_barrier()

        def body(it, carry):
            ea = pl.multiple_of(s * EPT + (2 * it) * CH, CH)
            eb = pl.multiple_of(s * EPT + (2 * it + 1) * CH, CH)
            load_idx(ea, sv0, dv0, av0)
            ga = pltpu.async_copy(y_hbm.at[sv0], rows0, gsem)
            load_idx(eb, sv1, dv1, av1)
            ga.wait()
            sa = pltpu.async_copy(rows0, acc_s.at[av0], ssem, add=True)
            gb = pltpu.async_copy(y_hbm.at[sv1], rows1, gsem)
            gb.wait()
            sb = pltpu.async_copy(rows1, acc_s.at[av1], ssem, add=True)
            sa.wait()
            sb.wait()
            return carry

        lax.fori_loop(0, EPT // (2 * CH), body, 0)
        plsc.subcore_barrier()

        pltpu.sync_copy(acc_s.at[pl.ds(a0, RPT)], rows0)
        pltpu.sync_copy(rows0, acc_hbm.at[pl.ds(r0, RPT)])
        plsc.subcore_barrier()


@functools.lru_cache(maxsize=None)
def _edge_kernel_fn():
    mesh = plsc.VectorSubcoreMesh(core_axis_name="c", subcore_axis_name="s")
    return pl.kernel(
        _edge_body,
        mesh=mesh,
        out_type=[
            jax.ShapeDtypeStruct((N2, 1, DH), jnp.float32),
            jax.ShapeDtypeStruct((N2, 1, DH), jnp.float32),
        ],
        scratch_types=[
            pltpu.VMEM((CH,), jnp.int32),
            pltpu.VMEM((CH,), jnp.int32),
            pltpu.VMEM((CH,), jnp.int32),
            pltpu.VMEM((CH,), jnp.int32),
            pltpu.VMEM((CH,), jnp.int32),
            pltpu.VMEM((CH,), jnp.int32),
            pltpu.VMEM((CH, 1, DH), jnp.float32),
            pltpu.VMEM((CH, 1, DH), jnp.float32),
            pltpu.SemaphoreType.DMA,
            pltpu.SemaphoreType.DMA,
            pltpu.MemorySpace.VMEM_SHARED((SH_ACC, 1, DH), jnp.float32),
        ],
    )


def _fin_body(alo_ref, ahi_ref, dinv_ref, b_ref, out_ref):
    dinv = dinv_ref[...]
    b = b_ref[...]
    out_ref[:, :DH] = jnp.maximum(alo_ref[...] * dinv + b[:, :DH], 0.0)
    out_ref[:, DH:] = jnp.maximum(ahi_ref[...] * dinv + b[:, DH:], 0.0)


def _finish(alo, ahi, dinv_col, b_row):
    BR = 1024
    return pl.pallas_call(
        _fin_body,
        grid=(N2 // BR,),
        in_specs=[
            pl.BlockSpec((BR, DH), lambda i: (i, 0)),
            pl.BlockSpec((BR, DH), lambda i: (i, 0)),
            pl.BlockSpec((BR, 1), lambda i: (i, 0)),
            pl.BlockSpec((1, D), lambda i: (0, 0)),
        ],
        out_specs=pl.BlockSpec((BR, D), lambda i: (i, 0)),
        out_shape=jax.ShapeDtypeStruct((N2, D), jnp.float32),
    )(alo, ahi, dinv_col, b_row)


def kernel(state, edge_index, edge_attributes, batch, W, b):
    src = edge_index[0]
    dst = edge_index[1]
    pad_e = E_PAD - E
    src_pad = jnp.concatenate([src, jnp.zeros((pad_e,), jnp.int32)])
    dst_pad = jnp.concatenate([dst, jnp.full((pad_e,), N2, jnp.int32)])
    state_pad = jnp.concatenate(
        [state, jnp.zeros((N2 - N, D), jnp.float32)], axis=0
    )

    degp = _deg_kernel_fn()(dst_pad).reshape(NSC, SH_DEG)[:, :N2]
    ylo, yhi, dinv_col = _matmul_scale(state_pad, W, degp.T)
    alo3, ahi3 = _edge_kernel_fn()(
        ylo.reshape(N2, 1, DH), yhi.reshape(N2, 1, DH), src_pad, dst_pad
    )
    out = _finish(
        alo3.reshape(N2, DH), ahi3.reshape(N2, DH), dinv_col, b.reshape(1, D)
    )
    return out[:N]


# speedup vs baseline: 4.9042x; 1.0201x over previous
import functools

import jax
import jax.numpy as jnp
from jax import lax
from jax.experimental import pallas as pl
from jax.experimental.pallas import tpu as pltpu
from jax.experimental.pallas import tpu_sc as plsc

N, E, D = 10000, 160000, 256
DH = D // 2
N2 = 10240
NSC = 2
NTILE = 16
CH = 320
EPT = 10240
E_PAD = EPT * NTILE
CHUNK_A = 1024
EPT_A = E_PAD // (NSC * NTILE)
SH_DEG = 10496
DEG_SLICE = SH_DEG // NTILE
RSC = N2 // NSC
RPT = RSC // NTILE
SH_ACC = RSC + 8 * NTILE


def _deg_body(dst_hbm, degp_hbm, dst_v, ones_v, stage_v, sem, deg_s):
    c = lax.axis_index("c")
    s = lax.axis_index("s")

    for i in range(CHUNK_A // 16):
        ones_v[pl.ds(16 * i, 16)] = jnp.full((16,), 1.0, jnp.float32)
    for i in range(DEG_SLICE // 16):
        stage_v[pl.ds(16 * i, 16)] = jnp.zeros((16,), jnp.float32)
    pltpu.sync_copy(stage_v, deg_s.at[pl.ds(s * DEG_SLICE, DEG_SLICE)])
    plsc.subcore_barrier()

    base = (s * NSC + c) * EPT_A

    def body(ch, carry):
        e0 = pl.multiple_of(base + ch * CHUNK_A, CHUNK_A)
        pltpu.sync_copy(dst_hbm.at[pl.ds(e0, CHUNK_A)], dst_v)
        pltpu.async_copy(ones_v, deg_s.at[dst_v], sem, add=True).wait()
        return carry

    lax.fori_loop(0, EPT_A // CHUNK_A, body, 0)
    plsc.subcore_barrier()
    pltpu.sync_copy(deg_s.at[pl.ds(s * DEG_SLICE, DEG_SLICE)], stage_v)
    pltpu.sync_copy(
        stage_v, degp_hbm.at[pl.ds(c * SH_DEG + s * DEG_SLICE, DEG_SLICE)]
    )


@functools.lru_cache(maxsize=None)
def _deg_kernel_fn():
    mesh = plsc.VectorSubcoreMesh(core_axis_name="c", subcore_axis_name="s")
    return pl.kernel(
        _deg_body,
        mesh=mesh,
        out_type=jax.ShapeDtypeStruct((NSC * SH_DEG,), jnp.float32),
        scratch_types=[
            pltpu.VMEM((CHUNK_A,), jnp.int32),
            pltpu.VMEM((CHUNK_A,), jnp.float32),
            pltpu.VMEM((DEG_SLICE,), jnp.float32),
            pltpu.SemaphoreType.DMA,
            pltpu.MemorySpace.VMEM_SHARED((SH_DEG,), jnp.float32),
        ],
    )


def _mm_body(state_ref, w_ref, degp_ref, ylo_ref, yhi_ref, dinv_ref):
    x = jnp.dot(state_ref[...], w_ref[...], preferred_element_type=jnp.float32)
    deg = 1.0 + degp_ref[:, 0:1] + degp_ref[:, 1:2]
    dinv = lax.rsqrt(deg)
    y = x * dinv
    ylo_ref[...] = y[:, :DH]
    yhi_ref[...] = y[:, DH:]
    dinv_ref[...] = dinv


def _matmul_scale(state_pad, W, degp_t):
    BR = 1024
    return pl.pallas_call(
        _mm_body,
        grid=(N2 // BR,),
        in_specs=[
            pl.BlockSpec((BR, D), lambda i: (i, 0)),
            pl.BlockSpec((D, D), lambda i: (0, 0)),
            pl.BlockSpec((BR, 2), lambda i: (i, 0)),
        ],
        out_specs=[
            pl.BlockSpec((BR, DH), lambda i: (i, 0)),
            pl.BlockSpec((BR, DH), lambda i: (i, 0)),
            pl.BlockSpec((BR, 1), lambda i: (i, 0)),
        ],
        out_shape=[
            jax.ShapeDtypeStruct((N2, DH), jnp.float32),
            jax.ShapeDtypeStruct((N2, DH), jnp.float32),
            jax.ShapeDtypeStruct((N2, 1), jnp.float32),
        ],
    )(state_pad, W, degp_t)


def _edge_body(ylo_hbm, yhi_hbm, src_hbm, dst_hbm, alo_hbm, ahi_hbm,
               sv0, dv0, av0, sv1, dv1, av1, rows0, rows1,
               gsem, ssem, acc_s):
    c = lax.axis_index("c")
    s = lax.axis_index("s")
    row_base = c * RSC
    iota = lax.iota(jnp.int32, 16)
    dum = RSC + s * 8 + (iota & 7)

    r0 = row_base + s * RPT
    a0 = s * RPT
    NCH = EPT // CH

    def load_idx(e0, sv, dv, av):
        pltpu.sync_copy(src_hbm.at[pl.ds(e0, CH)], sv)
        pltpu.sync_copy(dst_hbm.at[pl.ds(e0, CH)], dv)
        for i in range(CH // 16):
            dd = dv[pl.ds(16 * i, 16)]
            local = dd - row_base
            inr = (local >= 0) & (local < RSC)
            av[pl.ds(16 * i, 16)] = jnp.where(inr, local, dum)

    def drain_scatter():
        pltpu.make_async_copy(ylo_hbm.at[pl.ds(0, CH)], rows0, ssem).wait()

    for y_hbm, acc_hbm in ((ylo_hbm, alo_hbm), (yhi_hbm, ahi_hbm)):
        pltpu.sync_copy(y_hbm.at[pl.ds(r0, RPT)], rows0)
        pltpu.sync_copy(rows0, acc_s.at[pl.ds(a0, RPT)])
        plsc.subcore_barrier()

        def body(it, carry):
            ea = pl.multiple_of(s * EPT + 2 * it * CH, CH)
            eb = pl.multiple_of(s * EPT + (2 * it + 1) * CH, CH)

            @pl.when(it > 0)
            def _():
                drain_scatter()
            load_idx(ea, sv0, dv0, av0)
            ga = pltpu.async_copy(y_hbm.at[sv0], rows0, gsem)
            ga.wait()
            pltpu.async_copy(rows0, acc_s.at[av0], ssem, add=True)

            @pl.when(it > 0)
            def _():
                drain_scatter()
            load_idx(eb, sv1, dv1, av1)
            gb = pltpu.async_copy(y_hbm.at[sv1], rows1, gsem)
            gb.wait()
            pltpu.async_copy(rows1, acc_s.at[av1], ssem, add=True)
            return carry

        lax.fori_loop(0, NCH // 2, body, 0)
        drain_scatter()
        drain_scatter()
        plsc.subcore_barrier()

        pltpu.sync_copy(acc_s.at[pl.ds(a0, RPT)], rows0)
        pltpu.sync_copy(rows0, acc_hbm.at[pl.ds(r0, RPT)])
        plsc.subcore_barrier()


@functools.lru_cache(maxsize=None)
def _edge_kernel_fn():
    mesh = plsc.VectorSubcoreMesh(core_axis_name="c", subcore_axis_name="s")
    return pl.kernel(
        _edge_body,
        mesh=mesh,
        out_type=[
            jax.ShapeDtypeStruct((N2, 1, DH), jnp.float32),
            jax.ShapeDtypeStruct((N2, 1, DH), jnp.float32),
        ],
        scratch_types=[
            pltpu.VMEM((CH,), jnp.int32),
            pltpu.VMEM((CH,), jnp.int32),
            pltpu.VMEM((CH,), jnp.int32),
            pltpu.VMEM((CH,), jnp.int32),
            pltpu.VMEM((CH,), jnp.int32),
            pltpu.VMEM((CH,), jnp.int32),
            pltpu.VMEM((CH, 1, DH), jnp.float32),
            pltpu.VMEM((CH, 1, DH), jnp.float32),
            pltpu.SemaphoreType.DMA,
            pltpu.SemaphoreType.DMA,
            pltpu.MemorySpace.VMEM_SHARED((SH_ACC, 1, DH), jnp.float32),
        ],
    )


def _fin_body(alo_ref, ahi_ref, dinv_ref, b_ref, out_ref):
    dinv = dinv_ref[...]
    b = b_ref[...]
    out_ref[:, :DH] = jnp.maximum(alo_ref[...] * dinv + b[:, :DH], 0.0)
    out_ref[:, DH:] = jnp.maximum(ahi_ref[...] * dinv + b[:, DH:], 0.0)


def _finish(alo, ahi, dinv_col, b_row):
    BR = 1024
    return pl.pallas_call(
        _fin_body,
        grid=(N2 // BR,),
        in_specs=[
            pl.BlockSpec((BR, DH), lambda i: (i, 0)),
            pl.BlockSpec((BR, DH), lambda i: (i, 0)),
            pl.BlockSpec((BR, 1), lambda i: (i, 0)),
            pl.BlockSpec((1, D), lambda i: (0, 0)),
        ],
        out_specs=pl.BlockSpec((BR, D), lambda i: (i, 0)),
        out_shape=jax.ShapeDtypeStruct((N2, D), jnp.float32),
    )(alo, ahi, dinv_col, b_row)


def kernel(state, edge_index, edge_attributes, batch, W, b):
    src = edge_index[0]
    dst = edge_index[1]
    pad_e = E_PAD - E
    src_pad = jnp.concatenate([src, jnp.zeros((pad_e,), jnp.int32)])
    dst_pad = jnp.concatenate([dst, jnp.full((pad_e,), N2, jnp.int32)])
    state_pad = jnp.concatenate(
        [state, jnp.zeros((N2 - N, D), jnp.float32)], axis=0
    )

    degp = _deg_kernel_fn()(dst_pad).reshape(NSC, SH_DEG)[:, :N2]
    ylo, yhi, dinv_col = _matmul_scale(state_pad, W, degp.T)
    alo3, ahi3 = _edge_kernel_fn()(
        ylo.reshape(N2, 1, DH), yhi.reshape(N2, 1, DH), src_pad, dst_pad
    )
    out = _finish(
        alo3.reshape(N2, DH), ahi3.reshape(N2, DH), dinv_col, b.reshape(1, D)
    )
    return out[:N]


# speedup vs baseline: 4.9974x; 1.0190x over previous
import functools

import jax
import jax.numpy as jnp
from jax import lax
from jax.experimental import pallas as pl
from jax.experimental.pallas import tpu as pltpu
from jax.experimental.pallas import tpu_sc as plsc

N, E, D = 10000, 160000, 256
DH = D // 2
N2 = 10240
NSC = 2
NTILE = 16
CH = 320
EPT = 10240
E_PAD = EPT * NTILE
CHUNK_A = 1024
EPT_A = E_PAD // (NSC * NTILE)
SH_DEG = 10496
DEG_SLICE = SH_DEG // NTILE
RSC = N2 // NSC
RPT = RSC // NTILE
SH_ACC = RSC + 8 * NTILE


def _deg_body(dst_hbm, degp_hbm, dst_v, ones_v, stage_v, sem, deg_s):
    c = lax.axis_index("c")
    s = lax.axis_index("s")

    for i in range(CHUNK_A // 16):
        ones_v[pl.ds(16 * i, 16)] = jnp.full((16,), 1.0, jnp.float32)
    for i in range(DEG_SLICE // 16):
        stage_v[pl.ds(16 * i, 16)] = jnp.zeros((16,), jnp.float32)
    pltpu.sync_copy(stage_v, deg_s.at[pl.ds(s * DEG_SLICE, DEG_SLICE)])
    plsc.subcore_barrier()

    base = (s * NSC + c) * EPT_A

    def body(ch, carry):
        e0 = pl.multiple_of(base + ch * CHUNK_A, CHUNK_A)
        pltpu.sync_copy(dst_hbm.at[pl.ds(e0, CHUNK_A)], dst_v)
        pltpu.async_copy(ones_v, deg_s.at[dst_v], sem, add=True).wait()
        return carry

    lax.fori_loop(0, EPT_A // CHUNK_A, body, 0)
    plsc.subcore_barrier()
    pltpu.sync_copy(deg_s.at[pl.ds(s * DEG_SLICE, DEG_SLICE)], stage_v)
    pltpu.sync_copy(
        stage_v, degp_hbm.at[pl.ds(c * SH_DEG + s * DEG_SLICE, DEG_SLICE)]
    )


@functools.lru_cache(maxsize=None)
def _deg_kernel_fn():
    mesh = plsc.VectorSubcoreMesh(core_axis_name="c", subcore_axis_name="s")
    return pl.kernel(
        _deg_body,
        mesh=mesh,
        out_type=jax.ShapeDtypeStruct((NSC * SH_DEG,), jnp.float32),
        scratch_types=[
            pltpu.VMEM((CHUNK_A,), jnp.int32),
            pltpu.VMEM((CHUNK_A,), jnp.float32),
            pltpu.VMEM((DEG_SLICE,), jnp.float32),
            pltpu.SemaphoreType.DMA,
            pltpu.MemorySpace.VMEM_SHARED((SH_DEG,), jnp.float32),
        ],
    )


def _mm_body(state_ref, w_ref, degp_ref, ylo_ref, yhi_ref, dinv_ref):
    x = jnp.dot(state_ref[...], w_ref[...], preferred_element_type=jnp.float32)
    deg = 1.0 + degp_ref[:, 0:1] + degp_ref[:, 1:2]
    dinv = lax.rsqrt(deg)
    y = x * dinv
    ylo_ref[...] = y[:, :DH]
    yhi_ref[...] = y[:, DH:]
    dinv_ref[...] = dinv


def _matmul_scale(state_pad, W, degp_t):
    BR = 1024
    return pl.pallas_call(
        _mm_body,
        grid=(N2 // BR,),
        in_specs=[
            pl.BlockSpec((BR, D), lambda i: (i, 0)),
            pl.BlockSpec((D, D), lambda i: (0, 0)),
            pl.BlockSpec((BR, 2), lambda i: (i, 0)),
        ],
        out_specs=[
            pl.BlockSpec((BR, DH), lambda i: (i, 0)),
            pl.BlockSpec((BR, DH), lambda i: (i, 0)),
            pl.BlockSpec((BR, 1), lambda i: (i, 0)),
        ],
        out_shape=[
            jax.ShapeDtypeStruct((N2, DH), jnp.float32),
            jax.ShapeDtypeStruct((N2, DH), jnp.float32),
            jax.ShapeDtypeStruct((N2, 1), jnp.float32),
        ],
    )(state_pad, W, degp_t)


def _edge_body(ylo_hbm, yhi_hbm, src_hbm, dst_hbm, alo_hbm, ahi_hbm,
               sv0, dv0, av0, sv1, dv1, av1, rows0, rows1,
               gsem, ssem, isem, acc_s):
    c = lax.axis_index("c")
    s = lax.axis_index("s")
    row_base = c * RSC
    iota = lax.iota(jnp.int32, 16)
    dum = RSC + s * 8 + (iota & 7)

    r0 = row_base + s * RPT
    a0 = s * RPT
    NCH = EPT // CH

    def fetch_idx(e0, sv, dv):
        return (
            pltpu.async_copy(src_hbm.at[pl.ds(e0, CH)], sv, isem),
            pltpu.async_copy(dst_hbm.at[pl.ds(e0, CH)], dv, isem),
        )

    def remap(dv, av):
        for i in range(CH // 16):
            dd = dv[pl.ds(16 * i, 16)]
            local = dd - row_base
            inr = (local >= 0) & (local < RSC)
            av[pl.ds(16 * i, 16)] = jnp.where(inr, local, dum)

    def drain_scatter():
        pltpu.make_async_copy(ylo_hbm.at[pl.ds(0, CH)], rows0, ssem).wait()

    for y_hbm, acc_hbm in ((ylo_hbm, alo_hbm), (yhi_hbm, ahi_hbm)):
        pltpu.sync_copy(y_hbm.at[pl.ds(r0, RPT)], rows0)
        pltpu.sync_copy(rows0, acc_s.at[pl.ds(a0, RPT)])
        plsc.subcore_barrier()

        def body(it, carry):
            ea = pl.multiple_of(s * EPT + 2 * it * CH, CH)
            eb = pl.multiple_of(s * EPT + (2 * it + 1) * CH, CH)

            fa = fetch_idx(ea, sv0, dv0)
            fb = fetch_idx(eb, sv1, dv1)

            @pl.when(it > 0)
            def _():
                drain_scatter()
            for f in fa:
                f.wait()
            remap(dv0, av0)
            ga = pltpu.async_copy(y_hbm.at[sv0], rows0, gsem)
            ga.wait()
            pltpu.async_copy(rows0, acc_s.at[av0], ssem, add=True)

            @pl.when(it > 0)
            def _():
                drain_scatter()
            for f in fb:
                f.wait()
            remap(dv1, av1)
            gb = pltpu.async_copy(y_hbm.at[sv1], rows1, gsem)
            gb.wait()
            pltpu.async_copy(rows1, acc_s.at[av1], ssem, add=True)
            return carry

        lax.fori_loop(0, NCH // 2, body, 0)
        drain_scatter()
        drain_scatter()
        plsc.subcore_barrier()

        pltpu.sync_copy(acc_s.at[pl.ds(a0, RPT)], rows0)
        pltpu.sync_copy(rows0, acc_hbm.at[pl.ds(r0, RPT)])
        plsc.subcore_barrier()


@functools.lru_cache(maxsize=None)
def _edge_kernel_fn():
    mesh = plsc.VectorSubcoreMesh(core_axis_name="c", subcore_axis_name="s")
    return pl.kernel(
        _edge_body,
        mesh=mesh,
        out_type=[
            jax.ShapeDtypeStruct((N2, 1, DH), jnp.float32),
            jax.ShapeDtypeStruct((N2, 1, DH), jnp.float32),
        ],
        scratch_types=[
            pltpu.VMEM((CH,), jnp.int32),
            pltpu.VMEM((CH,), jnp.int32),
            pltpu.VMEM((CH,), jnp.int32),
            pltpu.VMEM((CH,), jnp.int32),
            pltpu.VMEM((CH,), jnp.int32),
            pltpu.VMEM((CH,), jnp.int32),
            pltpu.VMEM((CH, 1, DH), jnp.float32),
            pltpu.VMEM((CH, 1, DH), jnp.float32),
            pltpu.SemaphoreType.DMA,
            pltpu.SemaphoreType.DMA,
            pltpu.SemaphoreType.DMA,
            pltpu.MemorySpace.VMEM_SHARED((SH_ACC, 1, DH), jnp.float32),
        ],
    )


def _fin_body(alo_ref, ahi_ref, dinv_ref, b_ref, out_ref):
    dinv = dinv_ref[...]
    b = b_ref[...]
    out_ref[:, :DH] = jnp.maximum(alo_ref[...] * dinv + b[:, :DH], 0.0)
    out_ref[:, DH:] = jnp.maximum(ahi_ref[...] * dinv + b[:, DH:], 0.0)


def _finish(alo, ahi, dinv_col, b_row):
    BR = 1024
    return pl.pallas_call(
        _fin_body,
        grid=(N2 // BR,),
        in_specs=[
            pl.BlockSpec((BR, DH), lambda i: (i, 0)),
            pl.BlockSpec((BR, DH), lambda i: (i, 0)),
            pl.BlockSpec((BR, 1), lambda i: (i, 0)),
            pl.BlockSpec((1, D), lambda i: (0, 0)),
        ],
        out_specs=pl.BlockSpec((BR, D), lambda i: (i, 0)),
        out_shape=jax.ShapeDtypeStruct((N2, D), jnp.float32),
    )(alo, ahi, dinv_col, b_row)


def kernel(state, edge_index, edge_attributes, batch, W, b):
    src = edge_index[0]
    dst = edge_index[1]
    pad_e = E_PAD - E
    src_pad = jnp.concatenate([src, jnp.zeros((pad_e,), jnp.int32)])
    dst_pad = jnp.concatenate([dst, jnp.full((pad_e,), N2, jnp.int32)])
    state_pad = jnp.concatenate(
        [state, jnp.zeros((N2 - N, D), jnp.float32)], axis=0
    )

    degp = _deg_kernel_fn()(dst_pad).reshape(NSC, SH_DEG)[:, :N2]
    ylo, yhi, dinv_col = _matmul_scale(state_pad, W, degp.T)
    alo3, ahi3 = _edge_kernel_fn()(
        ylo.reshape(N2, 1, DH), yhi.reshape(N2, 1, DH), src_pad, dst_pad
    )
    out = _finish(
        alo3.reshape(N2, DH), ahi3.reshape(N2, DH), dinv_col, b.reshape(1, D)
    )
    return out[:N]
